# baseline (speedup 1.0000x reference)
import numpy as np
import jax
import jax.numpy as jnp
from jax import lax
from jax.experimental import pallas as pl
from jax.experimental.pallas import tpu as pltpu

B, C, H, W = 16, 384, 32, 32
S = 8
NH, NW = H // S, W // S
NWIN = NH * NW
NPIX = S * S
HW = H * W
K = int(NPIX * 0.5)
INV_LOG_N = 1.0 / float(np.log(float(NPIX)))


def _selectors():
    p = np.arange(HW)
    h, w = p // W, p % W
    win = (h // S) * NW + (w // S)
    tpos = (h % S) * S + (w % S)

    T2 = np.zeros((NPIX, HW), np.float32)
    T2[tpos, p] = 1.0
    E = np.zeros((NWIN, HW), np.float32)
    E[win, p] = 1.0
    UU = tpos.reshape(1, HW).astype(np.int32)
    return T2, E, UU


_T2, _E, _UU = _selectors()


def _mm(x, y, dims, prec):
    return lax.dot_general(x, y, (dims, ((), ())), precision=prec,
                           preferred_element_type=jnp.float32)


def _body(a_ref, b_ref, w_ref, bias_ref, fc1_ref, fc2_ref,
          t2_ref, e_ref, uu_ref, out_ref):
  hi = lax.Precision.HIGHEST
  lo = lax.Precision.DEFAULT
  for j in range(2):
    am = a_ref[j]
    bm = b_ref[j]
    t2 = t2_ref[...]
    e = e_ref[...]

    attn = _mm(w_ref[...], am, ((1,), (0,)), lo) + bias_ref[0, 0]

    blocksT = _mm(t2 * attn, e, ((1,), (1,)), hi)

    m = jnp.max(blocksT, axis=0, keepdims=True)
    ex = jnp.exp(blocksT - m)
    z = jnp.sum(ex, axis=0, keepdims=True)
    prob = ex / z
    entropy = -jnp.sum(prob * jnp.log(prob + 1e-8), axis=0, keepdims=True)
    norm_ent = jnp.maximum(0.1, entropy * INV_LOG_N)
    k_top = jnp.round(K * norm_ent)
    k_top_row = _mm(k_top, e, ((1,), (0,)), lo)

    p2 = _mm(blocksT, e, ((1,), (0,)), hi)
    p1 = jnp.sum(p2 * t2, axis=0, keepdims=True)

    vv = lax.broadcasted_iota(jnp.int32, (NPIX, HW), 0)
    tie = (p2 == p1) & (vv < uu_ref[...])
    rank_desc = jnp.sum(((p2 > p1) | tie).astype(jnp.float32),
                        axis=0, keepdims=True)
    rank_asc = jnp.sum(((p2 < p1) | tie).astype(jnp.float32),
                       axis=0, keepdims=True)
    sel = (rank_desc < k_top_row) | (rank_asc < (float(K) - k_top_row))
    mask_row = sel.astype(jnp.float32)

    pooled = _mm(mask_row, bm, ((1,), (1,)), lo) * (1.0 / HW)
    h1 = jnp.maximum(_mm(pooled, fc1_ref[...], ((1,), (1,)), lo), 0.0)
    out = _mm(h1, fc2_ref[...], ((1,), (1,)), lo)
    out_ref[j] = out


def _run(a_r, b_r, wv, bias, fc1, fc2, t2, e, uu):
    nb = a_r.shape[0]
    full = lambda shape: pl.BlockSpec(shape, lambda i: (0,) * len(shape))
    return pl.pallas_call(
        _body,
        grid=(nb // 2,),
        in_specs=[
            pl.BlockSpec((2, C, HW), lambda i: (i, 0, 0)),
            pl.BlockSpec((2, C, HW), lambda i: (i, 0, 0)),
            full((1, C)),
            full((1, 1)),
            full((C // 4, C)),
            full((C, C // 4)),
            full((NPIX, HW)),
            full((NWIN, HW)),
            full((1, HW)),
        ],
        out_specs=pl.BlockSpec((2, 1, C), lambda i: (i, 0, 0)),
        out_shape=jax.ShapeDtypeStruct((nb, 1, C), jnp.float32),
        compiler_params=pltpu.CompilerParams(
            dimension_semantics=("arbitrary",)),
    )(a_r, b_r, wv, bias, fc1, fc2, t2, e, uu)


def kernel(a, b, attn_w, attn_b, fc1_w, fc2_w):
    a_r = a.reshape(B, C, HW)
    b_r = b.reshape(B, C, HW)
    wv = attn_w.reshape(1, C)
    bias = attn_b.reshape(1, 1)
    fc1 = fc1_w.reshape(C // 4, C)
    fc2 = fc2_w.reshape(C, C // 4)
    consts = (jnp.asarray(_T2), jnp.asarray(_E), jnp.asarray(_UU))

    devs = jax.devices()
    if len(devs) >= 2:
        mesh = jax.sharding.Mesh(np.array(devs[:2]), ("d",))
        P = jax.sharding.PartitionSpec
        try:
            from jax import shard_map as _shard_map
        except ImportError:
            from jax.experimental.shard_map import shard_map as _shard_map
        run = _shard_map(
            _run, mesh=mesh,
            in_specs=(P("d"), P("d"), P(), P(), P(), P(), P(), P(), P()),
            out_specs=P("d"),
            check_vma=False,
        )
    else:
        run = _run
    out = run(a_r, b_r, wv, bias, fc1, fc2, *consts)
    return out.reshape(B, C, 1, 1)


# speedup vs baseline: 4.6713x; 4.6713x over previous
import numpy as np
import jax
import jax.numpy as jnp
from jax import lax
from jax.experimental import pallas as pl
from jax.experimental.pallas import tpu as pltpu

B, C, H, W = 16, 384, 32, 32
S = 8
NH, NW = H // S, W // S
NWIN = NH * NW
NPIX = S * S
HW = H * W
K = int(NPIX * 0.5)
INV_LOG_N = 1.0 / float(np.log(float(NPIX)))


def _selectors():
    p = np.arange(HW)
    h, w = p // W, p % W
    win = (h // S) * NW + (w // S)
    tpos = (h % S) * S + (w % S)

    T2 = np.zeros((NPIX, HW), np.float32)
    T2[tpos, p] = 1.0
    E = np.zeros((NWIN, HW), np.float32)
    E[win, p] = 1.0
    UU = tpos.reshape(1, HW).astype(np.int32)
    return T2, E, UU


_T2, _E, _UU = _selectors()


def _mm(x, y, dims, prec):
    return lax.dot_general(x, y, (dims, ((), ())), precision=prec,
                           preferred_element_type=jnp.float32)


def _body(a_ref, b_ref, w_ref, bias_ref, fc1_ref, fc2_ref,
          t2_ref, e_ref, uu_ref, out_ref):
  hi = lax.Precision.HIGHEST
  lo = lax.Precision.DEFAULT
  for j in range(2):
    am = a_ref[j]
    bm = b_ref[j]
    t2 = t2_ref[...]
    e = e_ref[...]

    attn = _mm(w_ref[...], am, ((1,), (0,)), lo) + bias_ref[0, 0]

    blocksT = _mm(t2 * attn, e, ((1,), (1,)), hi)

    m = jnp.max(blocksT, axis=0, keepdims=True)
    ex = jnp.exp(blocksT - m)
    z = jnp.sum(ex, axis=0, keepdims=True)
    prob = ex / z
    entropy = -jnp.sum(prob * jnp.log(prob + 1e-8), axis=0, keepdims=True)
    norm_ent = jnp.maximum(0.1, entropy * INV_LOG_N)
    k_top = jnp.round(K * norm_ent)
    k_top_row = _mm(k_top, e, ((1,), (0,)), lo)

    p2 = _mm(blocksT, e, ((1,), (0,)), hi)
    p1 = jnp.sum(p2 * t2, axis=0, keepdims=True)

    vv = lax.broadcasted_iota(jnp.int32, (NPIX, HW), 0)
    tie = (p2 == p1) & (vv < uu_ref[...])
    rank_desc = jnp.sum(((p2 > p1) | tie).astype(jnp.float32),
                        axis=0, keepdims=True)
    rank_asc = jnp.sum(((p2 < p1) | tie).astype(jnp.float32),
                       axis=0, keepdims=True)
    sel = (rank_desc < k_top_row) | (rank_asc < (float(K) - k_top_row))
    mask_row = sel.astype(jnp.float32)

    pooled = _mm(mask_row, bm, ((1,), (1,)), lo) * (1.0 / HW)
    h1 = jnp.maximum(_mm(pooled, fc1_ref[...], ((1,), (1,)), lo), 0.0)
    out = _mm(h1, fc2_ref[...], ((1,), (1,)), lo)
    out_ref[j] = out


def _run(a_r, b_r, wv, bias, fc1, fc2, t2, e, uu):
    nb = a_r.shape[0]
    full = lambda shape: pl.BlockSpec(shape, lambda i: (0,) * len(shape))
    return pl.pallas_call(
        _body,
        grid=(nb // 2,),
        in_specs=[
            pl.BlockSpec((2, C, HW), lambda i: (i, 0, 0)),
            pl.BlockSpec((2, C, HW), lambda i: (i, 0, 0)),
            full((1, C)),
            full((1, 1)),
            full((C // 4, C)),
            full((C, C // 4)),
            full((NPIX, HW)),
            full((NWIN, HW)),
            full((1, HW)),
        ],
        out_specs=pl.BlockSpec((2, 1, C), lambda i: (i, 0, 0)),
        out_shape=jax.ShapeDtypeStruct((nb, 1, C), jnp.float32),
        compiler_params=pltpu.CompilerParams(
            dimension_semantics=("arbitrary",)),
    )(a_r, b_r, wv, bias, fc1, fc2, t2, e, uu)


def kernel(a, b, attn_w, attn_b, fc1_w, fc2_w):
    a_r = a.reshape(B, C, HW)
    b_r = b.reshape(B, C, HW)
    wv = attn_w.reshape(1, C)
    bias = attn_b.reshape(1, 1)
    fc1 = fc1_w.reshape(C // 4, C)
    fc2 = fc2_w.reshape(C, C // 4)
    consts = (jnp.asarray(_T2), jnp.asarray(_E), jnp.asarray(_UU))
    out = _run(a_r, b_r, wv, bias, fc1, fc2, *consts)
    return out.reshape(B, C, 1, 1)
